# baseline (speedup 1.0000x reference)
import functools

import jax
import jax.numpy as jnp
from jax import lax
from jax.experimental import pallas as pl
from jax.experimental.pallas import tpu as pltpu
from jax.experimental.pallas import tpu_sc as plsc

N = 10000
E = 320000
D = 128
OUT = 128
NG = 64

NC = 2
NS = 16
NP = 10240
RPT = NP // NS
EPT = E // NS
EPTP = 20480
CHUNK = 64
NFULL = EPTP // CHUNK
NR = 4
IB = 20
NBLK = NFULL // IB

f32 = jnp.float32
i32 = jnp.int32

_MESH = plsc.VectorSubcoreMesh(core_axis_name="c", subcore_axis_name="s")


def _leaky(x):
    return jnp.where(x >= 0, x, 0.01 * x)


@functools.partial(
    pl.kernel,
    out_type=jax.ShapeDtypeStruct((NC * NP,), f32),
    mesh=_MESH,
    scratch_types=[
        pltpu.VMEM_SHARED((NP,), f32),
        pltpu.VMEM((RPT,), f32),
        pltpu.VMEM((CHUNK,), f32),
        [pltpu.VMEM((IB, 2, CHUNK), i32)] * 2,
        [pltpu.SemaphoreType.DMA] * 2,
        pltpu.SemaphoreType.DMA,
    ],
)
def _sc_deg(e6_hbm, deg_hbm, deg_sh, zb, ones_v, ibs, isems, ssem):
    c = lax.axis_index("c")
    s = lax.axis_index("s")

    def iload(k, buf, sem):
        pltpu.async_copy(e6_hbm.at[c, s, k], buf, sem)

    def idrain(buf, sem):
        pltpu.make_async_copy(e6_hbm.at[c, s, 0], buf, sem).wait()

    def block(buf):
        for i in range(IB):
            pltpu.async_copy(ones_v, deg_sh.at[buf.at[i, 1]], ssem, add=True)
        for i in range(IB):
            pltpu.make_async_copy(deg_hbm.at[pl.ds(0, CHUNK)], ones_v,
                                  ssem).wait()

    @pl.loop(0, RPT // 16)
    def _(i):
        zb[pl.ds(i * 16, 16)] = jnp.zeros((16,), f32)

    @pl.loop(0, CHUNK // 16)
    def _(i):
        ones_v[pl.ds(i * 16, 16)] = jnp.ones((16,), f32)

    iload(0, ibs[0], isems[0])
    pltpu.sync_copy(zb, deg_sh.at[pl.ds(s * RPT, RPT)])
    idrain(ibs[0], isems[0])
    plsc.subcore_barrier()

    iload(1, ibs[1], isems[1])
    block(ibs[0])

    @pl.loop(0, 7)
    def _(m):
        idrain(ibs[1], isems[1])
        iload(2 * m + 2, ibs[0], isems[0])
        block(ibs[1])
        idrain(ibs[0], isems[0])
        iload(2 * m + 3, ibs[1], isems[1])
        block(ibs[0])

    idrain(ibs[1], isems[1])
    block(ibs[1])

    plsc.subcore_barrier()
    pltpu.sync_copy(deg_sh.at[pl.ds(s * RPT, RPT)],
                    deg_hbm.at[pl.ds(c * NP + s * RPT, RPT)])


@functools.partial(
    pl.kernel,
    out_type=jax.ShapeDtypeStruct((NC * NP, D), f32),
    mesh=_MESH,
    scratch_types=[
        pltpu.VMEM_SHARED((NP, D), f32),
        [pltpu.VMEM((CHUNK, D), f32)] * NR,
        [pltpu.VMEM((IB, 2, CHUNK), i32)] * 2,
        [pltpu.SemaphoreType.DMA] * NR,
        [pltpu.SemaphoreType.DMA] * NR,
        [pltpu.SemaphoreType.DMA] * 2,
        pltpu.SemaphoreType.DMA,
    ],
)
def _sc_agg(hs_hbm, e6_hbm, acc_hbm,
            acc_sh, rows_v, ibs, gsems, ssems, isems, sem):
    c = lax.axis_index("c")
    s = lax.axis_index("s")
    rbase = c * NP + s * RPT

    def iload(k, bi):
        pltpu.async_copy(e6_hbm.at[c, s, k], ibs[bi], isems[bi])

    def idrain(bi):
        pltpu.make_async_copy(e6_hbm.at[c, s, 0], ibs[bi], isems[bi]).wait()

    def gissue(buf, i, b):
        pltpu.async_copy(hs_hbm.at[buf.at[i, 0]], rows_v[b], gsems[b])

    def gdrain(b):
        pltpu.make_async_copy(hs_hbm.at[pl.ds(0, CHUNK)], rows_v[b],
                              gsems[b]).wait()

    def sissue(buf, i, b):
        pltpu.async_copy(rows_v[b], acc_sh.at[buf.at[i, 1]], ssems[b],
                         add=True)

    def sdrain(b):
        pltpu.make_async_copy(hs_hbm.at[pl.ds(0, CHUNK)], rows_v[b],
                              ssems[b]).wait()

    def block(bi, next_k, first=False, last=False):
        buf, other = ibs[bi], ibs[1 - bi]
        for i in range(IB):
            b = i % NR
            b2 = (i + 2) % NR
            gdrain(b)
            if not (first and i < 2):
                sdrain(b2)
            if i == 1 and not last:
                iload(next_k, 1 - bi)
            sissue(buf, i, b)
            if i < IB - 2:
                gissue(buf, i + 2, b2)
            elif not last:
                if i == IB - 2:
                    idrain(1 - bi)
                gissue(other, i - (IB - 2), b2)

    iload(0, 0)
    pltpu.sync_copy(hs_hbm.at[pl.ds(rbase, RPT)],
                    acc_sh.at[pl.ds(s * RPT, RPT)])
    idrain(0)
    plsc.subcore_barrier()

    gissue(ibs[0], 0, 0)
    gissue(ibs[0], 1, 1)
    block(0, 1, first=True)

    @pl.loop(0, 7)
    def _(m):
        block(1, 2 * m + 2)
        block(0, 2 * m + 3)

    block(1, 0, last=True)
    sdrain((IB - 2) % NR)
    sdrain((IB - 1) % NR)

    plsc.subcore_barrier()
    pltpu.sync_copy(acc_sh.at[pl.ds(s * RPT, RPT)],
                    acc_hbm.at[pl.ds(rbase, RPT)])


RB = 1024
NRB = NP // RB


def _tc_mm(x_all, w_all):
    def body(x_ref, w_ref, h_ref):
        h_ref[0] = lax.dot_general(x_ref[0], w_ref[0], (((1,), (0,)), ((), ())),
                                   precision=lax.Precision.HIGHEST)

    return pl.pallas_call(
        body,
        grid=(NC, NRB),
        in_specs=[
            pl.BlockSpec((1, RB, D), lambda c, i: (c, i, 0)),
            pl.BlockSpec((1, D, D), lambda c, i: (c, 0, 0)),
        ],
        out_specs=pl.BlockSpec((1, RB, D), lambda c, i: (c, i, 0)),
        out_shape=jax.ShapeDtypeStruct((NC, NP, D), f32),
    )(x_all, w_all)


def _tc_scale(h_all, deg3):
    def body(h_ref, deg_ref, hs_ref):
        hs_ref[0] = h_ref[0] * lax.rsqrt(deg_ref[0] + 1.0)

    return pl.pallas_call(
        body,
        grid=(NC, NRB),
        in_specs=[
            pl.BlockSpec((1, RB, D), lambda c, i: (c, i, 0)),
            pl.BlockSpec((1, RB, 1), lambda c, i: (c, i, 0)),
        ],
        out_specs=pl.BlockSpec((1, RB, D), lambda c, i: (c, i, 0)),
        out_shape=jax.ShapeDtypeStruct((NC, NP, D), f32),
    )(h_all, deg3)


def _tc_poolhead(acc_all, deg3, bt_all, convb_all, fcw_all, fcb_all,
                 fc1_w, fc1_b, fc2_w, fc2_b, out_w, out_b):
    hp = lax.Precision.HIGHEST

    def dot(a, b):
        return lax.dot_general(a, b, (((1,), (0,)), ((), ())), precision=hp)

    def body(acc_ref, deg_ref, bt_ref, cb_ref, fw_ref, fb_ref,
             f1w_ref, f1b_ref, f2w_ref, f2b_ref, ow_ref, ob_ref, o_ref,
             ssum_ref, cnt_ref):
        c = pl.program_id(0)
        i = pl.program_id(1)
        dinv = lax.rsqrt(deg_ref[0] + 1.0)
        y = _leaky(acc_ref[0] * dinv + cb_ref[0])
        iota64 = lax.broadcasted_iota(i32, (NG, RB), 0)
        pt = (bt_ref[0] == iota64).astype(f32)
        ssum = dot(pt, y)
        cnt = dot(pt, jnp.ones((RB, D), f32))

        @pl.when(i == 0)
        def _():
            ssum_ref[c] = ssum
            cnt_ref[c] = cnt

        @pl.when(i > 0)
        def _():
            ssum_ref[c] += ssum
            cnt_ref[c] += cnt

        @pl.when((c == NC - 1) & (i == NRB - 1))
        def _():
            feats = []
            for cc in range(NC):
                mean = ssum_ref[cc] / jnp.maximum(cnt_ref[cc], 1.0)
                feats.append(_leaky(dot(mean, fw_ref[cc]) + fb_ref[cc]))
            xc = jnp.concatenate(feats, axis=1)
            a = _leaky(dot(xc, f1w_ref[...]) + f1b_ref[...])
            a = _leaky(dot(a, f2w_ref[...]) + f2b_ref[...])
            o_ref[...] = jax.nn.sigmoid(dot(a, ow_ref[...]) + ob_ref[...])

    fullspec = lambda shape: pl.BlockSpec(shape, lambda c, i: (0,) * len(shape))
    return pl.pallas_call(
        body,
        grid=(NC, NRB),
        in_specs=[
            pl.BlockSpec((1, RB, D), lambda c, i: (c, i, 0)),
            pl.BlockSpec((1, RB, 1), lambda c, i: (c, i, 0)),
            pl.BlockSpec((1, 1, RB), lambda c, i: (c, 0, i)),
            pl.BlockSpec((1, 1, D), lambda c, i: (c, 0, 0)),
            fullspec((NC, OUT, OUT)),
            fullspec((NC, 1, OUT)),
            fullspec((2 * OUT, 256)),
            fullspec((1, 256)),
            fullspec((256, 64)),
            fullspec((1, 64)),
            fullspec((64, 1)),
            fullspec((1, 1)),
        ],
        out_specs=pl.BlockSpec((NG, 1), lambda c, i: (0, 0)),
        out_shape=jax.ShapeDtypeStruct((NG, 1), f32),
        scratch_shapes=[
            pltpu.VMEM((NC, NG, D), f32),
            pltpu.VMEM((NC, NG, D), f32),
        ],
    )(acc_all, deg3, bt_all, convb_all, fcw_all, fcb_all,
      fc1_w, fc1_b, fc2_w, fc2_b, out_w, out_b)


def _pad_rows(x):
    return jnp.pad(x, ((0, NP - N), (0, 0)))


def kernel(pro1_x, pro1_edge_index, pro1_batch, pro2_x, pro2_edge_index,
           pro2_batch, pro1_conv_W, pro1_conv_b, pro1_fc_W, pro1_fc_b,
           pro2_conv_W, pro2_conv_b, pro2_fc_W, pro2_fc_b, fc1_W, fc1_b,
           fc2_W, fc2_b, out_W, out_b):
    x_all = jnp.stack([_pad_rows(pro1_x), _pad_rows(pro2_x)])
    w_all = jnp.stack([pro1_conv_W, pro2_conv_W])
    pad = ((0, 0), (0, 0), (0, EPTP - EPT))
    src3 = jnp.pad(
        jnp.stack([pro1_edge_index[0].astype(i32),
                   pro2_edge_index[0].astype(i32) + NP]).reshape(NC, NS, EPT),
        pad)
    dstpad = jnp.broadcast_to(
        N + jnp.arange(EPTP - EPT, dtype=i32) % (NP - N),
        (NC, NS, EPTP - EPT))
    dst3 = jnp.concatenate([
        jnp.stack([pro1_edge_index[1].astype(i32),
                   pro2_edge_index[1].astype(i32)]).reshape(NC, NS, EPT),
        dstpad], axis=2)
    e6 = jnp.stack([src3.reshape(NC, NS, NBLK, IB, CHUNK),
                    dst3.reshape(NC, NS, NBLK, IB, CHUNK)],
                   axis=4)

    deg_flat = _sc_deg(e6)
    h_all = _tc_mm(x_all, w_all)
    deg3 = deg_flat.reshape(NC, NP, 1)
    hs_all = _tc_scale(h_all, deg3)
    acc_flat = _sc_agg(hs_all.reshape(NC * NP, D), e6)

    bt_all = jnp.stack([
        jnp.pad(pro1_batch.astype(i32), (0, NP - N), constant_values=NG),
        jnp.pad(pro2_batch.astype(i32), (0, NP - N), constant_values=NG),
    ]).reshape(NC, 1, NP)
    convb_all = jnp.stack([pro1_conv_b.reshape(1, D),
                           pro2_conv_b.reshape(1, D)])
    fcw_all = jnp.stack([pro1_fc_W, pro2_fc_W])
    fcb_all = jnp.stack([pro1_fc_b.reshape(1, OUT),
                         pro2_fc_b.reshape(1, OUT)])

    return _tc_poolhead(acc_flat.reshape(NC, NP, D), deg3, bt_all,
                        convb_all, fcw_all, fcb_all,
                        fc1_W, fc1_b.reshape(1, 256), fc2_W,
                        fc2_b.reshape(1, 64), out_W, out_b.reshape(1, 1))


# speedup vs baseline: 2.4155x; 2.4155x over previous
import functools

import jax
import jax.numpy as jnp
from jax import lax
from jax.experimental import pallas as pl
from jax.experimental.pallas import tpu as pltpu
from jax.experimental.pallas import tpu_sc as plsc

N = 10000
E = 320000
D = 128
OUT = 128
NG = 64

NC = 2
NS = 16
NP = 10240
RPT = NP // NS
EPT = E // NS
CHUNK = 64
NFULL = EPT // CHUNK
TAIL = EPT - NFULL * CHUNK
NR = 4

f32 = jnp.float32
i32 = jnp.int32

_MESH = plsc.VectorSubcoreMesh(core_axis_name="c", subcore_axis_name="s")


def _leaky(x):
    return jnp.where(x >= 0, x, 0.01 * x)


IB = 24
NBLK = NFULL // IB


@functools.partial(
    pl.kernel,
    out_type=jax.ShapeDtypeStruct((NC * NP,), f32),
    mesh=_MESH,
    scratch_types=[
        pltpu.VMEM_SHARED((NP,), f32),
        pltpu.VMEM((RPT,), f32),
        pltpu.VMEM((CHUNK,), f32),
        [pltpu.VMEM((IB, 2, CHUNK), i32)] * 2,
        pltpu.VMEM((2, CHUNK), i32),
        [pltpu.SemaphoreType.DMA] * 2,
        pltpu.SemaphoreType.DMA,
    ],
)
def _sc_deg(e6_hbm, et_hbm, deg_hbm, deg_sh, zb, ones_v, ibs, tb, isems,
            ssem):
    c = lax.axis_index("c")
    s = lax.axis_index("s")

    def iload(k, buf, sem):
        pltpu.async_copy(e6_hbm.at[c, s, k], buf, sem)

    def idrain(buf, sem):
        pltpu.make_async_copy(e6_hbm.at[c, s, 0], buf, sem).wait()

    def scat(buf, i):
        pltpu.async_copy(ones_v, deg_sh.at[buf.at[i, 1]], ssem, add=True)

    def sdrain():
        pltpu.make_async_copy(deg_hbm.at[pl.ds(0, CHUNK)], ones_v,
                              ssem).wait()

    def block(buf):
        for i in range(IB):
            scat(buf, i)
        for i in range(IB):
            sdrain()

    @pl.loop(0, RPT // 16)
    def _(i):
        zb[pl.ds(i * 16, 16)] = jnp.zeros((16,), f32)

    @pl.loop(0, CHUNK // 16)
    def _(i):
        ones_v[pl.ds(i * 16, 16)] = jnp.ones((16,), f32)

    iload(0, ibs[0], isems[0])
    pltpu.sync_copy(et_hbm.at[c, s], tb)
    pltpu.sync_copy(zb, deg_sh.at[pl.ds(s * RPT, RPT)])
    idrain(ibs[0], isems[0])
    plsc.subcore_barrier()

    iload(1, ibs[1], isems[1])
    block(ibs[0])

    @pl.loop(0, 5)
    def _(m):
        idrain(ibs[1], isems[1])
        iload(2 * m + 2, ibs[0], isems[0])
        block(ibs[1])
        idrain(ibs[0], isems[0])
        iload(2 * m + 3, ibs[1], isems[1])
        block(ibs[0])

    idrain(ibs[1], isems[1])
    iload(12, ibs[0], isems[0])
    block(ibs[1])
    idrain(ibs[0], isems[0])
    block(ibs[0])

    pltpu.async_copy(ones_v, deg_sh.at[tb.at[1]], ssem, add=True)
    sdrain()

    plsc.subcore_barrier()
    pltpu.sync_copy(deg_sh.at[pl.ds(s * RPT, RPT)],
                    deg_hbm.at[pl.ds(c * NP + s * RPT, RPT)])


@functools.partial(
    pl.kernel,
    out_type=jax.ShapeDtypeStruct((NC * NP, D), f32),
    mesh=_MESH,
    scratch_types=[
        pltpu.VMEM_SHARED((NP, D), f32),
        [pltpu.VMEM((CHUNK, D), f32)] * NR,
        [pltpu.VMEM((IB, 2, CHUNK), i32)] * 2,
        pltpu.VMEM((2, CHUNK), i32),
        [pltpu.SemaphoreType.DMA] * NR,
        [pltpu.SemaphoreType.DMA] * NR,
        [pltpu.SemaphoreType.DMA] * 2,
        pltpu.SemaphoreType.DMA,
    ],
)
def _sc_agg(hs_hbm, e6_hbm, et_hbm, acc_hbm,
            acc_sh, rows_v, ibs, tb, gsems, ssems, isems, sem):
    c = lax.axis_index("c")
    s = lax.axis_index("s")
    rbase = c * NP + s * RPT

    def iload(k, bi):
        pltpu.async_copy(e6_hbm.at[c, s, k], ibs[bi], isems[bi])

    def idrain(bi):
        pltpu.make_async_copy(e6_hbm.at[c, s, 0], ibs[bi], isems[bi]).wait()

    def gissue(buf, i, b):
        pltpu.async_copy(hs_hbm.at[buf.at[i, 0]], rows_v[b], gsems[b])

    def gdrain(b):
        pltpu.make_async_copy(hs_hbm.at[pl.ds(0, CHUNK)], rows_v[b],
                              gsems[b]).wait()

    def sissue(buf, i, b):
        pltpu.async_copy(rows_v[b], acc_sh.at[buf.at[i, 1]], ssems[b],
                         add=True)

    def sdrain(b):
        pltpu.make_async_copy(hs_hbm.at[pl.ds(0, CHUNK)], rows_v[b],
                              ssems[b]).wait()

    def block(bi, next_k, first=False, last=False):
        buf, other = ibs[bi], ibs[1 - bi]
        for i in range(IB):
            b = i % NR
            b2 = (i + 2) % NR
            gdrain(b)
            if not (first and i < 2):
                sdrain(b2)
            if i == 1 and not last:
                iload(next_k, 1 - bi)
            sissue(buf, i, b)
            if i < IB - 2:
                gissue(buf, i + 2, b2)
            elif not last:
                if i == IB - 2:
                    idrain(1 - bi)
                gissue(other, i - (IB - 2), b2)

    iload(0, 0)
    pltpu.sync_copy(et_hbm.at[c, s], tb)
    pltpu.sync_copy(hs_hbm.at[pl.ds(rbase, RPT)],
                    acc_sh.at[pl.ds(s * RPT, RPT)])
    idrain(0)
    plsc.subcore_barrier()

    gissue(ibs[0], 0, 0)
    gissue(ibs[0], 1, 1)
    block(0, 1, first=True)

    @pl.loop(0, 5)
    def _(m):
        block(1, 2 * m + 2)
        block(0, 2 * m + 3)

    block(1, 12)
    block(0, 0, last=True)
    sdrain((IB - 2) % NR)
    sdrain((IB - 1) % NR)

    pltpu.async_copy(hs_hbm.at[tb.at[0]], rows_v[0], gsems[0])
    gdrain(0)
    pltpu.async_copy(rows_v[0], acc_sh.at[tb.at[1]], ssems[0], add=True)
    sdrain(0)

    plsc.subcore_barrier()
    pltpu.sync_copy(acc_sh.at[pl.ds(s * RPT, RPT)],
                    acc_hbm.at[pl.ds(rbase, RPT)])


RB = 1024
NRB = NP // RB


def _tc_pre(x_all, w_all, deg3):
    def body(x_ref, w_ref, deg_ref, hs_ref):
        dinv = lax.rsqrt(deg_ref[0] + 1.0)
        h = lax.dot_general(x_ref[0], w_ref[0], (((1,), (0,)), ((), ())),
                            precision=lax.Precision.HIGHEST)
        hs_ref[0] = h * dinv

    return pl.pallas_call(
        body,
        grid=(NC, NRB),
        in_specs=[
            pl.BlockSpec((1, RB, D), lambda c, i: (c, i, 0)),
            pl.BlockSpec((1, D, D), lambda c, i: (c, 0, 0)),
            pl.BlockSpec((1, RB, 1), lambda c, i: (c, i, 0)),
        ],
        out_specs=pl.BlockSpec((1, RB, D), lambda c, i: (c, i, 0)),
        out_shape=jax.ShapeDtypeStruct((NC, NP, D), f32),
    )(x_all, w_all, deg3)


def _tc_pool(acc_all, deg3, bt_all, convb_all):
    hp = lax.Precision.HIGHEST

    def body(acc_ref, deg_ref, bt_ref, cb_ref, ssum_ref, cnt_ref):
        i = pl.program_id(1)
        dinv = lax.rsqrt(deg_ref[0] + 1.0)
        y = _leaky(acc_ref[0] * dinv + cb_ref[0])
        iota64 = lax.broadcasted_iota(i32, (NG, RB), 0)
        pt = (bt_ref[0] == iota64).astype(f32)
        ssum = lax.dot_general(pt, y, (((1,), (0,)), ((), ())), precision=hp)
        cnt = lax.dot_general(pt, jnp.ones((RB, D), f32),
                              (((1,), (0,)), ((), ())), precision=hp)

        @pl.when(i == 0)
        def _():
            ssum_ref[0] = ssum
            cnt_ref[0] = cnt

        @pl.when(i > 0)
        def _():
            ssum_ref[0] += ssum
            cnt_ref[0] += cnt

    return pl.pallas_call(
        body,
        grid=(NC, NRB),
        in_specs=[
            pl.BlockSpec((1, RB, D), lambda c, i: (c, i, 0)),
            pl.BlockSpec((1, RB, 1), lambda c, i: (c, i, 0)),
            pl.BlockSpec((1, 1, RB), lambda c, i: (c, 0, i)),
            pl.BlockSpec((1, 1, D), lambda c, i: (c, 0, 0)),
        ],
        out_specs=[
            pl.BlockSpec((1, NG, D), lambda c, i: (c, 0, 0)),
            pl.BlockSpec((1, NG, D), lambda c, i: (c, 0, 0)),
        ],
        out_shape=[
            jax.ShapeDtypeStruct((NC, NG, D), f32),
            jax.ShapeDtypeStruct((NC, NG, D), f32),
        ],
    )(acc_all, deg3, bt_all, convb_all)


def _tc_head(ssum_all, cnt_all, fcw_all, fcb_all,
             fc1_w, fc1_b, fc2_w, fc2_b, out_w, out_b):
    hp = lax.Precision.HIGHEST

    def dot(a, b):
        return lax.dot_general(a, b, (((1,), (0,)), ((), ())), precision=hp)

    def body(ssum_ref, cnt_ref, fw_ref, fb_ref,
             f1w_ref, f1b_ref, f2w_ref, f2b_ref, ow_ref, ob_ref, o_ref):
        feats = []
        for c in range(NC):
            mean = ssum_ref[c] / jnp.maximum(cnt_ref[c], 1.0)
            feats.append(_leaky(dot(mean, fw_ref[c]) + fb_ref[c]))
        xc = jnp.concatenate(feats, axis=1)
        a = _leaky(dot(xc, f1w_ref[...]) + f1b_ref[...])
        a = _leaky(dot(a, f2w_ref[...]) + f2b_ref[...])
        o_ref[...] = jax.nn.sigmoid(dot(a, ow_ref[...]) + ob_ref[...])

    return pl.pallas_call(
        body,
        out_shape=jax.ShapeDtypeStruct((NG, 1), f32),
    )(ssum_all, cnt_all, fcw_all, fcb_all,
      fc1_w, fc1_b, fc2_w, fc2_b, out_w, out_b)


def _pad_rows(x):
    return jnp.pad(x, ((0, NP - N), (0, 0)))


def kernel(pro1_x, pro1_edge_index, pro1_batch, pro2_x, pro2_edge_index,
           pro2_batch, pro1_conv_W, pro1_conv_b, pro1_fc_W, pro1_fc_b,
           pro2_conv_W, pro2_conv_b, pro2_fc_W, pro2_fc_b, fc1_W, fc1_b,
           fc2_W, fc2_b, out_W, out_b):
    x_all = jnp.stack([_pad_rows(pro1_x), _pad_rows(pro2_x)])
    w_all = jnp.stack([pro1_conv_W, pro2_conv_W])
    src3 = jnp.stack([pro1_edge_index[0].astype(i32),
                      pro2_edge_index[0].astype(i32) + NP]).reshape(NC, NS,
                                                                    EPT)
    dst3 = jnp.stack([pro1_edge_index[1].astype(i32),
                      pro2_edge_index[1].astype(i32)]).reshape(NC, NS, EPT)
    nmain = NFULL * CHUNK
    e6 = jnp.stack([src3[:, :, :nmain].reshape(NC, NS, NBLK, IB, CHUNK),
                    dst3[:, :, :nmain].reshape(NC, NS, NBLK, IB, CHUNK)],
                   axis=4)
    pad = ((0, 0), (0, 0), (0, 0), (0, CHUNK - TAIL))
    et = jnp.stack([jnp.pad(src3[:, :, nmain:], pad[1:]),
                    jnp.pad(dst3[:, :, nmain:], pad[1:],
                            constant_values=NP - 1)],
                   axis=2)

    deg_flat = _sc_deg(e6, et)
    deg3 = deg_flat.reshape(NC, NP, 1)
    hs_all = _tc_pre(x_all, w_all, deg3)
    acc_flat = _sc_agg(hs_all.reshape(NC * NP, D), e6, et)

    bt_all = jnp.stack([
        jnp.pad(pro1_batch.astype(i32), (0, NP - N), constant_values=NG),
        jnp.pad(pro2_batch.astype(i32), (0, NP - N), constant_values=NG),
    ]).reshape(NC, 1, NP)
    convb_all = jnp.stack([pro1_conv_b.reshape(1, D),
                           pro2_conv_b.reshape(1, D)])
    fcw_all = jnp.stack([pro1_fc_W, pro2_fc_W])
    fcb_all = jnp.stack([pro1_fc_b.reshape(1, OUT),
                         pro2_fc_b.reshape(1, OUT)])

    ssum_all, cnt_all = _tc_pool(acc_flat.reshape(NC, NP, D), deg3, bt_all,
                                 convb_all)
    return _tc_head(ssum_all, cnt_all, fcw_all, fcb_all,
                    fc1_W, fc1_b.reshape(1, 256), fc2_W,
                    fc2_b.reshape(1, 64), out_W, out_b.reshape(1, 1))
